# baseline (device time: 431283 ns/iter reference)
import jax
import jax.numpy as jnp
from jax import lax
from jax.experimental import pallas as pl
from jax.experimental.pallas import tpu as pltpu

N_DEV = 32
HQ = 8
DH = 64
SCALE = 0.125
NEG = -1e30


def kernel(x, Wq, Wo, K_ext, V_ext):
    B, Sq, D = x.shape
    _, Skv, Hkv, _ = K_ext.shape
    R = B * Sq
    Rk = B * Skv

    x2d = x.reshape(R, D)
    kv = jnp.stack(
        [K_ext.reshape(Rk, Hkv * DH), V_ext.reshape(Rk, Hkv * DH)]
    )

    def body(x_ref, wq_ref, wo_ref, kv_ref, out_ref,
             comm_ref, send_sems, recv_sems):
        my = lax.axis_index("i")
        right = lax.rem(my + 1, N_DEV)
        left = lax.rem(my + N_DEV - 1, N_DEV)

        barrier_sem = pltpu.get_barrier_semaphore()
        for nbr in (left, right):
            pl.semaphore_signal(
                barrier_sem, inc=1,
                device_id=(nbr,), device_id_type=pl.DeviceIdType.MESH,
            )
        pl.semaphore_wait(barrier_sem, 2)

        q = jnp.dot(x_ref[:], wq_ref[:], preferred_element_type=jnp.float32)

        comm_ref[0] = kv_ref[:]

        row = lax.broadcasted_iota(jnp.int32, (R, Rk), 0)
        col = lax.broadcasted_iota(jnp.int32, (R, Rk), 1)
        ok = (row < Sq) == (col < Skv)

        def consume(kslab, vslab, state):
            new = []
            for h in range(HQ):
                m, l, a = state[h]
                qh = q[:, h * DH:(h + 1) * DH]
                kh = kslab[:, h * DH:(h + 1) * DH]
                vh = vslab[:, h * DH:(h + 1) * DH]
                s = lax.dot_general(
                    qh, kh, (((1,), (1,)), ((), ())),
                    preferred_element_type=jnp.float32,
                ) * SCALE
                s = jnp.where(ok, s, NEG)
                m_new = jnp.maximum(m, jnp.max(s, axis=1, keepdims=True))
                alpha = jnp.exp(m - m_new)
                p = jnp.exp(s - m_new)
                l_new = l * alpha + jnp.sum(p, axis=1, keepdims=True)
                a_new = a * alpha + jnp.dot(
                    p, vh, preferred_element_type=jnp.float32
                )
                new.append((m_new, l_new, a_new))
            return new

        state = [
            (jnp.full((R, 1), NEG, jnp.float32),
             jnp.zeros((R, 1), jnp.float32),
             jnp.zeros((R, DH), jnp.float32))
            for _ in range(HQ)
        ]

        def hop(h, state):
            slot = lax.rem(h, 2)
            rdma = pltpu.make_async_remote_copy(
                src_ref=comm_ref.at[slot],
                dst_ref=comm_ref.at[1 - slot],
                send_sem=send_sems.at[slot],
                recv_sem=recv_sems.at[1 - slot],
                device_id=(right,),
                device_id_type=pl.DeviceIdType.MESH,
            )
            rdma.start()
            state = consume(comm_ref[slot, 0], comm_ref[slot, 1], state)
            rdma.wait()
            return state

        state = lax.fori_loop(0, N_DEV - 1, hop, state)
        last = (N_DEV - 1) % 2
        state = consume(comm_ref[last, 0], comm_ref[last, 1], state)

        o = jnp.concatenate([a / l for (_, l, a) in state], axis=1)
        out_ref[:] = jnp.dot(o, wo_ref[:], preferred_element_type=jnp.float32)

    out2d = pl.pallas_call(
        body,
        out_shape=jax.ShapeDtypeStruct((R, D), jnp.float32),
        in_specs=[pl.BlockSpec(memory_space=pltpu.VMEM)] * 4,
        out_specs=pl.BlockSpec(memory_space=pltpu.VMEM),
        scratch_shapes=[
            pltpu.VMEM((2, 2, Rk, Hkv * DH), jnp.float32),
            pltpu.SemaphoreType.DMA((2,)),
            pltpu.SemaphoreType.DMA((2,)),
        ],
        compiler_params=pltpu.CompilerParams(collective_id=0),
    )(x2d, Wq, Wo, kv)
    return out2d.reshape(B, Sq, D)


# device time: 401613 ns/iter; 1.0739x vs baseline; 1.0739x over previous
import jax
import jax.numpy as jnp
from jax import lax
from jax.experimental import pallas as pl
from jax.experimental.pallas import tpu as pltpu

N_DEV = 32
HOPS = N_DEV // 2
NSLOT = 4
HQ = 8
DH = 64
SCALE = 0.125
NEG = -1e30


def kernel(x, Wq, Wo, K_ext, V_ext):
    B, Sq, D = x.shape
    _, Skv, Hkv, _ = K_ext.shape
    R = B * Sq
    Rk = B * Skv

    x2d = x.reshape(R, D)
    kv = jnp.stack(
        [K_ext.reshape(Rk, Hkv * DH), V_ext.reshape(Rk, Hkv * DH)]
    )

    def body(x_ref, wq_ref, wo_ref, kv_ref, out_ref,
             commr_ref, comml_ref, sendr, recvr, sendl, recvl):
        my = lax.axis_index("i")
        right = lax.rem(my + 1, N_DEV)
        left = lax.rem(my + N_DEV - 1, N_DEV)

        barrier_sem = pltpu.get_barrier_semaphore()
        for nbr in (left, right):
            pl.semaphore_signal(
                barrier_sem, inc=1,
                device_id=(nbr,), device_id_type=pl.DeviceIdType.MESH,
            )
        pl.semaphore_wait(barrier_sem, 2)

        q = jnp.dot(x_ref[:], wq_ref[:], preferred_element_type=jnp.float32)

        commr_ref[0] = kv_ref[:]
        comml_ref[0] = kv_ref[:]

        row = lax.broadcasted_iota(jnp.int32, (R, Rk), 0)
        col = lax.broadcasted_iota(jnp.int32, (R, Rk), 1)
        ok = (row < Sq) == (col < Skv)

        def consume(kslab, vslab, state):
            new = []
            for h in range(HQ):
                m, l, a = state[h]
                qh = q[:, h * DH:(h + 1) * DH]
                kh = kslab[:, h * DH:(h + 1) * DH]
                vh = vslab[:, h * DH:(h + 1) * DH]
                s = lax.dot_general(
                    qh, kh, (((1,), (1,)), ((), ())),
                    preferred_element_type=jnp.float32,
                ) * SCALE
                s = jnp.where(ok, s, NEG)
                m_new = jnp.maximum(m, jnp.max(s, axis=1, keepdims=True))
                alpha = jnp.exp(m - m_new)
                p = jnp.exp(s - m_new)
                l_new = l * alpha + jnp.sum(p, axis=1, keepdims=True)
                a_new = a * alpha + jnp.dot(
                    p, vh, preferred_element_type=jnp.float32
                )
                new.append((m_new, l_new, a_new))
            return new

        def send(comm_ref, s_sems, r_sems, i, dst):
            slot = i % NSLOT
            rdma = pltpu.make_async_remote_copy(
                src_ref=comm_ref.at[slot],
                dst_ref=comm_ref.at[(i + 1) % NSLOT],
                send_sem=s_sems.at[slot],
                recv_sem=r_sems.at[(i + 1) % NSLOT],
                device_id=(dst,),
                device_id_type=pl.DeviceIdType.MESH,
            )
            rdma.start()
            return rdma

        state = [
            (jnp.full((R, 1), NEG, jnp.float32),
             jnp.zeros((R, 1), jnp.float32),
             jnp.zeros((R, DH), jnp.float32))
            for _ in range(HQ)
        ]

        rr = send(commr_ref, sendr, recvr, 0, right)
        rl = send(comml_ref, sendl, recvl, 0, left)
        state = consume(kv_ref[0], kv_ref[1], state)
        rr.wait()
        rl.wait()

        def hop(i, state):
            s = lax.rem(i, NSLOT)
            rr = send(commr_ref, sendr, recvr, i, right)
            rl = send(comml_ref, sendl, recvl, i, left)
            state = consume(commr_ref[s, 0], commr_ref[s, 1], state)
            state = consume(comml_ref[s, 0], comml_ref[s, 1], state)
            rr.wait()
            rl.wait()
            return state

        state = lax.fori_loop(1, HOPS - 1, hop, state)

        s = (HOPS - 1) % NSLOT
        rr = send(commr_ref, sendr, recvr, HOPS - 1, right)
        state = consume(commr_ref[s, 0], commr_ref[s, 1], state)
        state = consume(comml_ref[s, 0], comml_ref[s, 1], state)
        rr.wait()

        s = HOPS % NSLOT
        state = consume(commr_ref[s, 0], commr_ref[s, 1], state)

        o = jnp.concatenate([a / l for (_, l, a) in state], axis=1)
        out_ref[:] = jnp.dot(o, wo_ref[:], preferred_element_type=jnp.float32)

    out2d = pl.pallas_call(
        body,
        out_shape=jax.ShapeDtypeStruct((R, D), jnp.float32),
        in_specs=[pl.BlockSpec(memory_space=pltpu.VMEM)] * 4,
        out_specs=pl.BlockSpec(memory_space=pltpu.VMEM),
        scratch_shapes=[
            pltpu.VMEM((NSLOT, 2, Rk, Hkv * DH), jnp.float32),
            pltpu.VMEM((NSLOT, 2, Rk, Hkv * DH), jnp.float32),
            pltpu.SemaphoreType.DMA((NSLOT,)),
            pltpu.SemaphoreType.DMA((NSLOT,)),
            pltpu.SemaphoreType.DMA((NSLOT,)),
            pltpu.SemaphoreType.DMA((NSLOT,)),
        ],
        compiler_params=pltpu.CompilerParams(collective_id=0),
    )(x2d, Wq, Wo, kv)
    return out2d.reshape(B, Sq, D)


# device time: 227484 ns/iter; 1.8959x vs baseline; 1.7655x over previous
import jax
import jax.numpy as jnp
import numpy as np
from jax import lax
from jax.experimental import pallas as pl
from jax.experimental.pallas import tpu as pltpu

N_DEV = 32

_SNAKE8 = [(0, 0), (1, 0), (1, 1), (0, 1), (0, 2), (1, 2), (1, 3), (0, 3)]
_COORD_TO_LOGICAL = {
    (xy[0], xy[1], z): z * 8 + q
    for z in range(4)
    for q, xy in enumerate(_SNAKE8)
}
_PATH16 = [
    (y, z)
    for z in range(4)
    for y in (range(4) if z % 2 == 0 else range(3, -1, -1))
]
_CYCLE = (
    [_COORD_TO_LOGICAL[(0, y, z)] for (y, z) in _PATH16]
    + [_COORD_TO_LOGICAL[(1, y, z)] for (y, z) in reversed(_PATH16)]
)
assert sorted(_CYCLE) == list(range(N_DEV))
_NEXT = np.zeros(N_DEV, np.int32)
_PREV = np.zeros(N_DEV, np.int32)
for _k, _p in enumerate(_CYCLE):
    _NEXT[_p] = _CYCLE[(_k + 1) % N_DEV]
    _PREV[_p] = _CYCLE[(_k - 1) % N_DEV]
HOPS = N_DEV // 2
NSLOT = 4
HQ = 8
DH = 64
SCALE = 0.125
NEG = -1e30


def kernel(x, Wq, Wo, K_ext, V_ext):
    B, Sq, D = x.shape
    _, Skv, Hkv, _ = K_ext.shape
    R = B * Sq
    Rk = B * Skv

    x2d = x.reshape(R, D)
    kv = jnp.stack(
        [K_ext.reshape(Rk, Hkv * DH), V_ext.reshape(Rk, Hkv * DH)]
    )

    def body(next_ref, prev_ref, x_ref, wq_ref, wo_ref, kv_ref, out_ref,
             commr_ref, comml_ref, sendr, recvr, sendl, recvl):
        my = lax.axis_index("i")
        right = next_ref[my]
        left = prev_ref[my]

        barrier_sem = pltpu.get_barrier_semaphore()
        for nbr in (left, right):
            pl.semaphore_signal(
                barrier_sem, inc=1,
                device_id=(nbr,), device_id_type=pl.DeviceIdType.MESH,
            )
        pl.semaphore_wait(barrier_sem, 2)

        q = jnp.dot(x_ref[:], wq_ref[:], preferred_element_type=jnp.float32)

        commr_ref[0] = kv_ref[:]
        comml_ref[0] = kv_ref[:]

        row = lax.broadcasted_iota(jnp.int32, (R, Rk), 0)
        col = lax.broadcasted_iota(jnp.int32, (R, Rk), 1)
        ok = (row < Sq) == (col < Skv)

        def consume(kslab, vslab, state):
            new = []
            for h in range(HQ):
                m, l, a = state[h]
                qh = q[:, h * DH:(h + 1) * DH]
                kh = kslab[:, h * DH:(h + 1) * DH]
                vh = vslab[:, h * DH:(h + 1) * DH]
                s = lax.dot_general(
                    qh, kh, (((1,), (1,)), ((), ())),
                    preferred_element_type=jnp.float32,
                ) * SCALE
                s = jnp.where(ok, s, NEG)
                m_new = jnp.maximum(m, jnp.max(s, axis=1, keepdims=True))
                alpha = jnp.exp(m - m_new)
                p = jnp.exp(s - m_new)
                l_new = l * alpha + jnp.sum(p, axis=1, keepdims=True)
                a_new = a * alpha + jnp.dot(
                    p, vh, preferred_element_type=jnp.float32
                )
                new.append((m_new, l_new, a_new))
            return new

        def send(comm_ref, s_sems, r_sems, i, dst):
            slot = i % NSLOT
            rdma = pltpu.make_async_remote_copy(
                src_ref=comm_ref.at[slot],
                dst_ref=comm_ref.at[(i + 1) % NSLOT],
                send_sem=s_sems.at[slot],
                recv_sem=r_sems.at[(i + 1) % NSLOT],
                device_id=(dst,),
                device_id_type=pl.DeviceIdType.MESH,
            )
            rdma.start()
            return rdma

        state = [
            (jnp.full((R, 1), NEG, jnp.float32),
             jnp.zeros((R, 1), jnp.float32),
             jnp.zeros((R, DH), jnp.float32))
            for _ in range(HQ)
        ]

        rr = send(commr_ref, sendr, recvr, 0, right)
        rl = send(comml_ref, sendl, recvl, 0, left)
        state = consume(kv_ref[0], kv_ref[1], state)
        rr.wait()
        rl.wait()

        def hop(i, state):
            s = lax.rem(i, NSLOT)
            rr = send(commr_ref, sendr, recvr, i, right)
            rl = send(comml_ref, sendl, recvl, i, left)
            state = consume(commr_ref[s, 0], commr_ref[s, 1], state)
            state = consume(comml_ref[s, 0], comml_ref[s, 1], state)
            rr.wait()
            rl.wait()
            return state

        state = lax.fori_loop(1, HOPS - 1, hop, state)

        s = (HOPS - 1) % NSLOT
        rr = send(commr_ref, sendr, recvr, HOPS - 1, right)
        state = consume(commr_ref[s, 0], commr_ref[s, 1], state)
        state = consume(comml_ref[s, 0], comml_ref[s, 1], state)
        rr.wait()

        s = HOPS % NSLOT
        state = consume(commr_ref[s, 0], commr_ref[s, 1], state)

        o = jnp.concatenate([a / l for (_, l, a) in state], axis=1)
        out_ref[:] = jnp.dot(o, wo_ref[:], preferred_element_type=jnp.float32)

    out2d = pl.pallas_call(
        body,
        out_shape=jax.ShapeDtypeStruct((R, D), jnp.float32),
        in_specs=[pl.BlockSpec(memory_space=pltpu.SMEM)] * 2
        + [pl.BlockSpec(memory_space=pltpu.VMEM)] * 4,
        out_specs=pl.BlockSpec(memory_space=pltpu.VMEM),
        scratch_shapes=[
            pltpu.VMEM((NSLOT, 2, Rk, Hkv * DH), jnp.float32),
            pltpu.VMEM((NSLOT, 2, Rk, Hkv * DH), jnp.float32),
            pltpu.SemaphoreType.DMA((NSLOT,)),
            pltpu.SemaphoreType.DMA((NSLOT,)),
            pltpu.SemaphoreType.DMA((NSLOT,)),
            pltpu.SemaphoreType.DMA((NSLOT,)),
        ],
        compiler_params=pltpu.CompilerParams(collective_id=0),
    )(jnp.asarray(_NEXT), jnp.asarray(_PREV), x2d, Wq, Wo, kv)
    return out2d.reshape(B, Sq, D)


# device time: 227087 ns/iter; 1.8992x vs baseline; 1.0017x over previous
import jax
import jax.numpy as jnp
import numpy as np
from jax import lax
from jax.experimental import pallas as pl
from jax.experimental.pallas import tpu as pltpu

N_DEV = 32

_SNAKE8 = [(0, 0), (1, 0), (1, 1), (0, 1), (0, 2), (1, 2), (1, 3), (0, 3)]
_COORD_TO_LOGICAL = {
    (xy[0], xy[1], z): z * 8 + q
    for z in range(4)
    for q, xy in enumerate(_SNAKE8)
}
_PATH16 = [
    (y, z)
    for z in range(4)
    for y in (range(4) if z % 2 == 0 else range(3, -1, -1))
]
_CYCLE = (
    [_COORD_TO_LOGICAL[(0, y, z)] for (y, z) in _PATH16]
    + [_COORD_TO_LOGICAL[(1, y, z)] for (y, z) in reversed(_PATH16)]
)
assert sorted(_CYCLE) == list(range(N_DEV))
_NEXT = np.zeros(N_DEV, np.int32)
_PREV = np.zeros(N_DEV, np.int32)
for _k, _p in enumerate(_CYCLE):
    _NEXT[_p] = _CYCLE[(_k + 1) % N_DEV]
    _PREV[_p] = _CYCLE[(_k - 1) % N_DEV]
HOPS = N_DEV // 2
NSLOT = 4
HQ = 8
DH = 64
SCALE = 0.125
NEG = -1e30


def kernel(x, Wq, Wo, K_ext, V_ext):
    B, Sq, D = x.shape
    _, Skv, Hkv, _ = K_ext.shape
    R = B * Sq
    Rk = B * Skv

    x2d = x.reshape(R, D)
    kv = jnp.stack(
        [K_ext.reshape(Rk, Hkv * DH), V_ext.reshape(Rk, Hkv * DH)]
    )

    def body(next_ref, prev_ref, x_ref, wq_ref, wo_ref, kv_ref, out_ref,
             commr_ref, comml_ref, sendr, recvr, sendl, recvl):
        my = lax.axis_index("i")
        right = next_ref[my]
        left = prev_ref[my]

        barrier_sem = pltpu.get_barrier_semaphore()
        for nbr in (left, right):
            pl.semaphore_signal(
                barrier_sem, inc=1,
                device_id=(nbr,), device_id_type=pl.DeviceIdType.MESH,
            )
        pl.semaphore_wait(barrier_sem, 2)

        row = lax.broadcasted_iota(jnp.int32, (R, Rk), 0)
        col = lax.broadcasted_iota(jnp.int32, (R, Rk), 1)
        ok = (row < Sq) == (col < Skv)

        def consume(kslab, vslab, state):
            new = []
            for h in range(HQ):
                m, l, a = state[h]
                qh = q[:, h * DH:(h + 1) * DH]
                kh = kslab[:, h * DH:(h + 1) * DH]
                vh = vslab[:, h * DH:(h + 1) * DH]
                s = lax.dot_general(
                    qh, kh, (((1,), (1,)), ((), ())),
                    preferred_element_type=jnp.float32,
                ) * SCALE
                s = jnp.where(ok, s, NEG)
                m_new = jnp.maximum(m, jnp.max(s, axis=1, keepdims=True))
                alpha = jnp.exp(m - m_new)
                p = jnp.exp(s - m_new)
                l_new = l * alpha + jnp.sum(p, axis=1, keepdims=True)
                a_new = a * alpha + jnp.dot(
                    p, vh, preferred_element_type=jnp.float32
                )
                new.append((m_new, l_new, a_new))
            return new

        def send(comm_ref, s_sems, r_sems, i, dst):
            slot = i % NSLOT
            rdma = pltpu.make_async_remote_copy(
                src_ref=comm_ref.at[slot],
                dst_ref=comm_ref.at[(i + 1) % NSLOT],
                send_sem=s_sems.at[slot],
                recv_sem=r_sems.at[(i + 1) % NSLOT],
                device_id=(dst,),
                device_id_type=pl.DeviceIdType.MESH,
            )
            rdma.start()
            return rdma

        state = [
            (jnp.full((R, 1), NEG, jnp.float32),
             jnp.zeros((R, 1), jnp.float32),
             jnp.zeros((R, DH), jnp.float32))
            for _ in range(HQ)
        ]

        def send0(comm_ref, s_sems, r_sems, dst):
            rdma = pltpu.make_async_remote_copy(
                src_ref=kv_ref,
                dst_ref=comm_ref.at[1],
                send_sem=s_sems.at[0],
                recv_sem=r_sems.at[1],
                device_id=(dst,),
                device_id_type=pl.DeviceIdType.MESH,
            )
            rdma.start()
            return rdma

        rr = send0(commr_ref, sendr, recvr, right)
        rl = send0(comml_ref, sendl, recvl, left)
        q = jnp.dot(x_ref[:], wq_ref[:], preferred_element_type=jnp.float32)
        state = consume(kv_ref[0], kv_ref[1], state)
        rr.wait()
        rl.wait()

        def hop(i, state):
            s = lax.rem(i, NSLOT)
            rr = send(commr_ref, sendr, recvr, i, right)
            rl = send(comml_ref, sendl, recvl, i, left)
            state = consume(commr_ref[s, 0], commr_ref[s, 1], state)
            state = consume(comml_ref[s, 0], comml_ref[s, 1], state)
            rr.wait()
            rl.wait()
            return state

        state = lax.fori_loop(1, HOPS - 1, hop, state)

        s = (HOPS - 1) % NSLOT
        rr = send(commr_ref, sendr, recvr, HOPS - 1, right)
        state = consume(commr_ref[s, 0], commr_ref[s, 1], state)
        state = consume(comml_ref[s, 0], comml_ref[s, 1], state)
        rr.wait()

        s = HOPS % NSLOT
        state = consume(commr_ref[s, 0], commr_ref[s, 1], state)

        o = jnp.concatenate([a / l for (_, l, a) in state], axis=1)
        out_ref[:] = jnp.dot(o, wo_ref[:], preferred_element_type=jnp.float32)

    out2d = pl.pallas_call(
        body,
        out_shape=jax.ShapeDtypeStruct((R, D), jnp.float32),
        in_specs=[pl.BlockSpec(memory_space=pltpu.SMEM)] * 2
        + [pl.BlockSpec(memory_space=pltpu.VMEM)] * 4,
        out_specs=pl.BlockSpec(memory_space=pltpu.VMEM),
        scratch_shapes=[
            pltpu.VMEM((NSLOT, 2, Rk, Hkv * DH), jnp.float32),
            pltpu.VMEM((NSLOT, 2, Rk, Hkv * DH), jnp.float32),
            pltpu.SemaphoreType.DMA((NSLOT,)),
            pltpu.SemaphoreType.DMA((NSLOT,)),
            pltpu.SemaphoreType.DMA((NSLOT,)),
            pltpu.SemaphoreType.DMA((NSLOT,)),
        ],
        compiler_params=pltpu.CompilerParams(collective_id=0),
    )(jnp.asarray(_NEXT), jnp.asarray(_PREV), x2d, Wq, Wo, kv)
    return out2d.reshape(B, Sq, D)


# device time: 192608 ns/iter; 2.2392x vs baseline; 1.1790x over previous
import jax
import jax.numpy as jnp
import numpy as np
from jax import lax
from jax.experimental import pallas as pl
from jax.experimental.pallas import tpu as pltpu

N_DEV = 32
HOPS = N_DEV // 2
NSLOT = 4
HQ = 8
DH = 64
SCALE = 0.125
NEG = -1e30

_SNAKE8 = [(0, 0), (1, 0), (1, 1), (0, 1), (0, 2), (1, 2), (1, 3), (0, 3)]
_COORD_TO_LOGICAL = {
    (xy[0], xy[1], z): z * 8 + q
    for z in range(4)
    for q, xy in enumerate(_SNAKE8)
}
_PATH16 = [
    (y, z)
    for z in range(4)
    for y in (range(4) if z % 2 == 0 else range(3, -1, -1))
]
_CYCLE = (
    [_COORD_TO_LOGICAL[(0, y, z)] for (y, z) in _PATH16]
    + [_COORD_TO_LOGICAL[(1, y, z)] for (y, z) in reversed(_PATH16)]
)
assert sorted(_CYCLE) == list(range(N_DEV))
_NEXT = np.zeros(N_DEV, np.int32)
_PREV = np.zeros(N_DEV, np.int32)
for _k, _p in enumerate(_CYCLE):
    _NEXT[_p] = _CYCLE[(_k + 1) % N_DEV]
    _PREV[_p] = _CYCLE[(_k - 1) % N_DEV]


def kernel(x, Wq, Wo, K_ext, V_ext):
    B, Sq, D = x.shape
    _, Skv, Hkv, _ = K_ext.shape
    R = B * Sq
    Rk = B * Skv

    x2d = x.reshape(R, D)
    k2d = K_ext.reshape(Rk, Hkv * DH)
    v2d = V_ext.reshape(Rk, Hkv * DH)

    def body(next_ref, prev_ref, x_ref, wq_ref, wo_ref, k_ref, v_ref,
             out_ref, brk, brv, blk, blv,
             sems_send, sems_recv):
        my = lax.axis_index("i")
        right = next_ref[my]
        left = prev_ref[my]

        barrier_sem = pltpu.get_barrier_semaphore()
        for nbr in (left, right):
            pl.semaphore_signal(
                barrier_sem, inc=1,
                device_id=(nbr,), device_id_type=pl.DeviceIdType.MESH,
            )
        pl.semaphore_wait(barrier_sem, 2)

        bufs = (brk, brv, blk, blv)
        srcs0 = (k_ref, v_ref, k_ref, v_ref)
        dsts = (right, right, left, left)

        def rdma(ss, j, src_ref=None):
            buf = bufs[ss]
            return pltpu.make_async_remote_copy(
                src_ref=buf.at[j % NSLOT] if src_ref is None else src_ref,
                dst_ref=buf.at[(j + 1) % NSLOT],
                send_sem=sems_send.at[ss, j % NSLOT],
                recv_sem=sems_recv.at[ss, (j + 1) % NSLOT],
                device_id=(dsts[ss],),
                device_id_type=pl.DeviceIdType.MESH,
            )

        for ss in range(4):
            rdma(ss, 0, src_ref=srcs0[ss]).start()

        q = jnp.dot(x_ref[:], wq_ref[:], preferred_element_type=jnp.float32)

        row = lax.broadcasted_iota(jnp.int32, (R, Rk), 0)
        col = lax.broadcasted_iota(jnp.int32, (R, Rk), 1)
        ok = (row < Sq) == (col < Skv)

        def consume(kslab, vslab, state):
            new = []
            for h in range(HQ):
                m, l, a = state[h]
                qh = q[:, h * DH:(h + 1) * DH]
                kh = kslab[:, h * DH:(h + 1) * DH]
                vh = vslab[:, h * DH:(h + 1) * DH]
                s = lax.dot_general(
                    qh, kh, (((1,), (1,)), ((), ())),
                    preferred_element_type=jnp.float32,
                ) * SCALE
                s = jnp.where(ok, s, NEG)
                m_new = jnp.maximum(m, jnp.max(s, axis=1, keepdims=True))
                alpha = jnp.exp(m - m_new)
                p = jnp.exp(s - m_new)
                l_new = l * alpha + jnp.sum(p, axis=1, keepdims=True)
                a_new = a * alpha + jnp.dot(
                    p, vh, preferred_element_type=jnp.float32
                )
                new.append((m_new, l_new, a_new))
            return new

        state = [
            (jnp.full((R, 1), NEG, jnp.float32),
             jnp.zeros((R, 1), jnp.float32),
             jnp.zeros((R, DH), jnp.float32))
            for _ in range(HQ)
        ]

        def step(i, state, do_consume_left=True, send_left=True,
                 send_right=True, local=False, wait_send0=False):
            if local:
                state = consume(k_ref[:], v_ref[:], state)
            else:
                s = i % NSLOT
                state = consume(brk[s], brv[s], state)
                if do_consume_left:
                    state = consume(blk[s], blv[s], state)
            for ss in range(4):
                is_left = ss >= 2
                rdma(ss, i).wait_recv()
                if (not is_left and send_right) or (is_left and send_left):
                    rdma(ss, i + 1).start()
            for ss in range(4):
                src0 = srcs0[ss] if wait_send0 else None
                rdma(ss, i, src_ref=src0).wait_send()
            return state

        state = step(0, state, do_consume_left=False, local=True,
                     wait_send0=True)

        def hop(i, state):
            return step(i, state)

        state = lax.fori_loop(1, HOPS - 2, hop, state)

        state = step(HOPS - 2, state, send_left=False)

        s = (HOPS - 1) % NSLOT
        state = consume(brk[s], brv[s], state)
        state = consume(blk[s], blv[s], state)
        for ss in (0, 1):
            rdma(ss, HOPS - 1).wait_recv()
            rdma(ss, HOPS - 1).wait_send()

        s = HOPS % NSLOT
        state = consume(brk[s], brv[s], state)

        o = jnp.concatenate([a / l for (_, l, a) in state], axis=1)
        out_ref[:] = jnp.dot(o, wo_ref[:], preferred_element_type=jnp.float32)

    out2d = pl.pallas_call(
        body,
        out_shape=jax.ShapeDtypeStruct((R, D), jnp.float32),
        in_specs=[pl.BlockSpec(memory_space=pltpu.SMEM)] * 2
        + [pl.BlockSpec(memory_space=pltpu.VMEM)] * 5,
        out_specs=pl.BlockSpec(memory_space=pltpu.VMEM),
        scratch_shapes=[
            pltpu.VMEM((NSLOT, Rk, Hkv * DH), jnp.float32),
            pltpu.VMEM((NSLOT, Rk, Hkv * DH), jnp.float32),
            pltpu.VMEM((NSLOT, Rk, Hkv * DH), jnp.float32),
            pltpu.VMEM((NSLOT, Rk, Hkv * DH), jnp.float32),
            pltpu.SemaphoreType.DMA((4, NSLOT)),
            pltpu.SemaphoreType.DMA((4, NSLOT)),
        ],
        compiler_params=pltpu.CompilerParams(collective_id=0),
    )(jnp.asarray(_NEXT), jnp.asarray(_PREV), x2d, Wq, Wo, k2d, v2d)
    return out2d.reshape(B, Sq, D)
